# baseline (speedup 1.0000x reference)
import functools

import jax
import jax.numpy as jnp
from jax import lax
from jax.experimental import pallas as pl
from jax.experimental.pallas import tpu as pltpu
from jax.experimental.pallas import tpu_sc as plsc

NC = 2
NS = 16
NW = NC * NS

_MESH = dict(core_axis_name="c", subcore_axis_name="s")


def _gather_call(B, V, D, S0, S1):
    CH = 128
    NB = 4
    nq = (B // CH) // NW
    b_per_w = B // NW
    njb = S0 // CH

    @functools.partial(
        pl.kernel,
        mesh=plsc.VectorSubcoreMesh(**_MESH),
        out_type=jax.ShapeDtypeStruct((S1, D // 8, njb, 8, CH), jnp.float32),
        scratch_types=[
            pltpu.VMEM((b_per_w,), jnp.int32),
            pltpu.VMEM((NB, CH), jnp.int32),
            pltpu.VMEM((NB, CH, 128), jnp.float32),
            pltpu.VMEM((2, D // 8, 8, CH), jnp.float32),
            pltpu.SemaphoreType.DMA,
            pltpu.SemaphoreType.DMA,
            pltpu.SemaphoreType.DMA,
            pltpu.SemaphoreType.DMA,
            pltpu.SemaphoreType.DMA,
            pltpu.SemaphoreType.DMA,
        ],
        compiler_params=pltpu.CompilerParams(use_tc_tiling_on_sc=True, needs_layout_passes=False),
    )
    def c2(idx_hbm, wp_hbm, o5_hbm, idx_v, ihc, g, t, g0, g1, g2, g3, w0, w1):
        wid = lax.axis_index("s") * NC + lax.axis_index("c")
        q0 = wid * nq
        gsem = (g0, g1, g2, g3)
        wsem = (w0, w1)
        iot = lax.iota(jnp.int32, 16)

        pltpu.sync_copy(idx_hbm.at[pl.ds(wid * b_per_w, b_per_w)], idx_v)

        def fire_gather(qt, b):
            for m in range(8):
                raw = idx_v[pl.ds(qt * CH + 16 * m, 16)]
                ihc[b, pl.ds(16 * m, 16)] = lax.shift_right_logical(raw, 1)
            pltpu.async_copy(wp_hbm.at[ihc.at[b]], g.at[b], gsem[b])

        def wait_gather(b):
            pltpu.make_async_copy(
                wp_hbm.at[pl.ds(0, CH)], g.at[b], gsem[b]
            ).wait()

        def fire_write(q, tb):
            s = q // njb
            jb = q % njb
            pltpu.async_copy(t.at[tb], o5_hbm.at[s, :, jb], wsem[tb])

        def wait_write(tb):
            pltpu.make_async_copy(
                o5_hbm.at[0, :, 0], t.at[tb], wsem[tb]
            ).wait()

        cvs = [iot + 16 * m for m in range(8)]

        def transpose(qt, b, tb):
            poff = [
                lax.shift_left(idx_v[pl.ds(qt * CH + 16 * m, 16)] & 1, 6)
                for m in range(8)
            ]

            @pl.loop(0, D)
            def _(i):
                dvec = ((iot + i) & 15) + (i & (D - 16))
                kvec = dvec >> 3
                rvec = dvec & 7
                for m in range(8):
                    v = plsc.load_gather(g.at[b], [cvs[m], dvec + poff[m]])
                    plsc.store_scatter(t.at[tb], [kvec, rvec, cvs[m]], v)

        for k in range(NB - 1):
            fire_gather(k, k)

        @pl.loop(0, nq // NB)
        def _(tt):
            for b in range(NB):
                qt = tt * NB + b
                tb = b % 2
                wait_gather(b)

                nxt = qt + NB - 1

                @pl.when(nxt < nq)
                def _():
                    fire_gather(nxt, (b + NB - 1) % NB)

                if b < 2:
                    @pl.when(tt > 0)
                    def _():
                        wait_write(tb)
                else:
                    wait_write(tb)

                transpose(qt, b, tb)
                fire_write(q0 + qt, tb)

        wait_write(0)
        wait_write(1)

    return c2


@jax.jit
def kernel(x, W):
    B0, S = x.shape
    V, D = W.shape
    B = B0 * S
    xf = x.T.reshape(B)
    wp = W.reshape(V // 2, 2 * D)
    o5 = _gather_call(B, V, D, B0, S)(xf, wp)
    return o5.transpose(2, 4, 0, 1, 3).reshape(B0, S, D)


# speedup vs baseline: 1.2040x; 1.2040x over previous
import functools

import jax
import jax.numpy as jnp
from jax import lax
from jax.experimental import pallas as pl
from jax.experimental.pallas import tpu as pltpu
from jax.experimental.pallas import tpu_sc as plsc

NC = 2
NS = 16
NW = NC * NS

_MESH = dict(core_axis_name="c", subcore_axis_name="s")


def _gather_call(B, V, D, S0, S1):
    CH = 128
    NB = 4
    nq = (B // CH) // NW
    b_per_w = B // NW
    njb = S0 // CH

    @functools.partial(
        pl.kernel,
        mesh=plsc.VectorSubcoreMesh(**_MESH),
        out_type=jax.ShapeDtypeStruct((S1, D // 8, njb, 8, CH), jnp.float32),
        scratch_types=[
            pltpu.VMEM((b_per_w,), jnp.int32),
            pltpu.VMEM((NB, CH), jnp.int32),
            pltpu.VMEM((NB, CH, 128), jnp.float32),
            pltpu.VMEM((2, D // 8, 8, CH), jnp.float32),
            pltpu.SemaphoreType.DMA,
            pltpu.SemaphoreType.DMA,
            pltpu.SemaphoreType.DMA,
            pltpu.SemaphoreType.DMA,
            pltpu.SemaphoreType.DMA,
            pltpu.SemaphoreType.DMA,
        ],
        compiler_params=pltpu.CompilerParams(use_tc_tiling_on_sc=True, needs_layout_passes=False),
    )
    def c2(idx_hbm, wp_hbm, o5_hbm, idx_v, ihc, g, t, g0, g1, g2, g3, w0, w1):
        wid = lax.axis_index("s") * NC + lax.axis_index("c")
        q0 = wid * nq
        gsem = (g0, g1, g2, g3)
        wsem = (w0, w1)
        iot = lax.iota(jnp.int32, 16)

        pltpu.sync_copy(idx_hbm.at[pl.ds(wid * b_per_w, b_per_w)], idx_v)

        def fire_gather(qt, b):
            for m in range(8):
                raw = idx_v[pl.ds(qt * CH + 16 * m, 16)]
                ihc[b, pl.ds(16 * m, 16)] = lax.shift_right_logical(raw, 1)
            pltpu.async_copy(wp_hbm.at[ihc.at[b]], g.at[b], gsem[b])

        def wait_gather(b):
            pltpu.make_async_copy(
                wp_hbm.at[pl.ds(0, CH)], g.at[b], gsem[b]
            ).wait()

        def fire_write(q, tb):
            s = q // njb
            jb = q % njb
            pltpu.async_copy(t.at[tb], o5_hbm.at[s, :, jb], wsem[tb])

        def wait_write(tb):
            pltpu.make_async_copy(
                o5_hbm.at[0, :, 0], t.at[tb], wsem[tb]
            ).wait()

        def transpose(qt, b, tb):
            poff = [
                lax.shift_left(idx_v[pl.ds(qt * CH + 16 * m, 16)] & 1, 6)
                for m in range(8)
            ]

            cvs = [iot + 16 * m for m in range(8)]

            @plsc.parallel_loop(0, D // 16, unroll=2)
            def _(dblk):
                d0 = dblk * 16
                rotv = iot
                for j in range(16):
                    dvec = rotv + d0
                    kvec = dvec >> 3
                    rvec = dvec & 7
                    for mp in range(4):
                        v0 = plsc.load_gather(
                            g.at[b], [cvs[2 * mp], dvec + poff[2 * mp]]
                        )
                        v1 = plsc.load_gather(
                            g.at[b], [cvs[2 * mp + 1], dvec + poff[2 * mp + 1]]
                        )
                        plsc.store_scatter(t.at[tb], [kvec, rvec, cvs[2 * mp]], v0)
                        plsc.store_scatter(
                            t.at[tb], [kvec, rvec, cvs[2 * mp + 1]], v1
                        )
                    if j < 15:
                        rotv = (rotv + 1) & 15

        for k in range(NB - 1):
            fire_gather(k, k)

        @pl.loop(0, nq // NB)
        def _(tt):
            for b in range(NB):
                qt = tt * NB + b
                tb = b % 2
                wait_gather(b)

                nxt = qt + NB - 1

                @pl.when(nxt < nq)
                def _():
                    fire_gather(nxt, (b + NB - 1) % NB)

                if b < 2:
                    @pl.when(tt > 0)
                    def _():
                        wait_write(tb)
                else:
                    wait_write(tb)

                transpose(qt, b, tb)
                fire_write(q0 + qt, tb)

        wait_write(0)
        wait_write(1)

    return c2


@jax.jit
def kernel(x, W):
    B0, S = x.shape
    V, D = W.shape
    B = B0 * S
    xf = x.T.reshape(B)
    wp = W.reshape(V // 2, 2 * D)
    o5 = _gather_call(B, V, D, B0, S)(xf, wp)
    return o5.transpose(2, 4, 0, 1, 3).reshape(B0, S, D)


# speedup vs baseline: 1.9786x; 1.6434x over previous
import functools

import jax
import jax.numpy as jnp
from jax import lax
from jax.experimental import pallas as pl
from jax.experimental.pallas import tpu as pltpu
from jax.experimental.pallas import tpu_sc as plsc

NC = 2
NS = 16
NW = NC * NS

_MESH = dict(core_axis_name="c", subcore_axis_name="s")


def _relayout_call(V, D):
    n_full = V // 128
    per_w = n_full // NW
    n_extra = n_full - per_w * NW

    @functools.partial(
        pl.kernel,
        mesh=plsc.VectorSubcoreMesh(**_MESH),
        out_type=jax.ShapeDtypeStruct((V // 2, 128), jnp.float32),
        scratch_types=[
            pltpu.VMEM((2, D, 128), jnp.float32),
            pltpu.VMEM((2, D, 128), jnp.float32),
            pltpu.SemaphoreType.DMA,
            pltpu.SemaphoreType.DMA,
            pltpu.SemaphoreType.DMA,
            pltpu.SemaphoreType.DMA,
        ],
        compiler_params=pltpu.CompilerParams(use_tc_tiling_on_sc=True, needs_layout_passes=False),
    )
    def c1(wt_hbm, wtail_hbm, wp_hbm, ibuf, obuf, r0, r1, w0, w1):
        wid = lax.axis_index("s") * NC + lax.axis_index("c")
        j0 = wid * per_w
        rsem = (r0, r1)
        wsem = (w0, w1)
        iot = lax.iota(jnp.int32, 16)
        cvs = [iot + 16 * m for m in range(8)]
        rvs = [lax.shift_right_logical(iot, 1) + 8 * m for m in range(8)]
        pc = lax.shift_left(iot & 1, 6)

        def fire_read(j, b):
            pltpu.async_copy(
                wt_hbm.at[:, pl.ds(j * 128, 128)], ibuf.at[b], rsem[b]
            )

        def wait_read(b):
            pltpu.make_async_copy(
                wt_hbm.at[:, pl.ds(0, 128)], ibuf.at[b], rsem[b]
            ).wait()

        def fire_write(j, b):
            pltpu.async_copy(
                obuf.at[b], wp_hbm.at[pl.ds(j * 64, 64)], wsem[b]
            )

        def wait_write(b):
            pltpu.make_async_copy(
                wp_hbm.at[pl.ds(0, 64)], obuf.at[b], wsem[b]
            ).wait()

        def transpose(b):
            @plsc.parallel_loop(0, D // 16, unroll=2)
            def _(dblk):
                d0 = dblk * 16
                rotv = iot
                for j in range(16):
                    dvec = rotv + d0
                    colv = dvec + pc
                    for mp in range(4):
                        v0 = plsc.load_gather(ibuf.at[b], [dvec, cvs[2 * mp]])
                        v1 = plsc.load_gather(
                            ibuf.at[b], [dvec, cvs[2 * mp + 1]]
                        )
                        plsc.store_scatter(obuf.at[b], [rvs[2 * mp], colv], v0)
                        plsc.store_scatter(
                            obuf.at[b], [rvs[2 * mp + 1], colv], v1
                        )
                    if j < 15:
                        rotv = (rotv + 1) & 15

        fire_read(j0, 0)
        fire_read(j0 + 1, 1)

        @pl.loop(0, per_w // 2)
        def _(t):
            j = j0 + 2 * t
            for b in range(2):
                wait_read(b)

                @pl.when(t > 0)
                def _():
                    wait_write(b)

                transpose(b)
                fire_write(j + b, b)
                nxt = j + b + 2

                @pl.when(nxt < j0 + per_w)
                def _():
                    fire_read(nxt, b)

        wait_write(0)
        wait_write(1)

        @pl.when(wid >= NW - n_extra)
        def _():
            j = n_full - n_extra + (wid - (NW - n_extra))
            fire_read(j, 0)
            wait_read(0)
            transpose(0)
            fire_write(j, 0)
            wait_write(0)

        @pl.when(wid == 0)
        def _():
            pltpu.sync_copy(wtail_hbm, ibuf.at[0])
            transpose(0)
            pltpu.sync_copy(
                obuf.at[0, pl.ds(0, 32)], wp_hbm.at[pl.ds(V // 2 - 32, 32)]
            )

    return c1


def _gather_call(B, V, D, S0, S1):
    CH = 128
    NB = 4
    nq = (B // CH) // NW
    b_per_w = B // NW
    njb = S0 // CH

    @functools.partial(
        pl.kernel,
        mesh=plsc.VectorSubcoreMesh(**_MESH),
        out_type=jax.ShapeDtypeStruct((S1, D // 8, njb, 8, CH), jnp.float32),
        scratch_types=[
            pltpu.VMEM((b_per_w,), jnp.int32),
            pltpu.VMEM((NB, CH), jnp.int32),
            pltpu.VMEM((NB, CH, 128), jnp.float32),
            pltpu.VMEM((2, D // 8, 8, CH), jnp.float32),
            pltpu.SemaphoreType.DMA,
            pltpu.SemaphoreType.DMA,
            pltpu.SemaphoreType.DMA,
            pltpu.SemaphoreType.DMA,
            pltpu.SemaphoreType.DMA,
            pltpu.SemaphoreType.DMA,
        ],
        compiler_params=pltpu.CompilerParams(use_tc_tiling_on_sc=True, needs_layout_passes=False),
    )
    def c2(idx_hbm, wp_hbm, o5_hbm, idx_v, ihc, g, t, g0, g1, g2, g3, w0, w1):
        wid = lax.axis_index("s") * NC + lax.axis_index("c")
        q0 = wid * nq
        gsem = (g0, g1, g2, g3)
        wsem = (w0, w1)
        iot = lax.iota(jnp.int32, 16)

        pltpu.sync_copy(idx_hbm.at[pl.ds(wid * b_per_w, b_per_w)], idx_v)

        def fire_gather(qt, b):
            for m in range(8):
                raw = idx_v[pl.ds(qt * CH + 16 * m, 16)]
                ihc[b, pl.ds(16 * m, 16)] = lax.shift_right_logical(raw, 1)
            pltpu.async_copy(wp_hbm.at[ihc.at[b]], g.at[b], gsem[b])

        def wait_gather(b):
            pltpu.make_async_copy(
                wp_hbm.at[pl.ds(0, CH)], g.at[b], gsem[b]
            ).wait()

        def fire_write(q, tb):
            s = q // njb
            jb = q % njb
            pltpu.async_copy(t.at[tb], o5_hbm.at[s, :, jb], wsem[tb])

        def wait_write(tb):
            pltpu.make_async_copy(
                o5_hbm.at[0, :, 0], t.at[tb], wsem[tb]
            ).wait()

        def transpose(qt, b, tb):
            poff = [
                lax.shift_left(idx_v[pl.ds(qt * CH + 16 * m, 16)] & 1, 6)
                for m in range(8)
            ]

            cvs = [iot + 16 * m for m in range(8)]

            @plsc.parallel_loop(0, D // 16, unroll=2)
            def _(dblk):
                d0 = dblk * 16
                rotv = iot
                for j in range(16):
                    dvec = rotv + d0
                    kvec = dvec >> 3
                    rvec = dvec & 7
                    for mp in range(4):
                        v0 = plsc.load_gather(
                            g.at[b], [cvs[2 * mp], dvec + poff[2 * mp]]
                        )
                        v1 = plsc.load_gather(
                            g.at[b], [cvs[2 * mp + 1], dvec + poff[2 * mp + 1]]
                        )
                        plsc.store_scatter(t.at[tb], [kvec, rvec, cvs[2 * mp]], v0)
                        plsc.store_scatter(
                            t.at[tb], [kvec, rvec, cvs[2 * mp + 1]], v1
                        )
                    if j < 15:
                        rotv = (rotv + 1) & 15

        for k in range(NB - 1):
            fire_gather(k, k)

        @pl.loop(0, nq // NB)
        def _(tt):
            for b in range(NB):
                qt = tt * NB + b
                tb = b % 2
                wait_gather(b)

                nxt = qt + NB - 1

                @pl.when(nxt < nq)
                def _():
                    fire_gather(nxt, (b + NB - 1) % NB)

                if b < 2:
                    @pl.when(tt > 0)
                    def _():
                        wait_write(tb)
                else:
                    wait_write(tb)

                transpose(qt, b, tb)
                fire_write(q0 + qt, tb)

        wait_write(0)
        wait_write(1)

    return c2


@jax.jit
def kernel(x, W):
    B0, S = x.shape
    V, D = W.shape
    B = B0 * S
    xf = x.T.reshape(B)
    wt = W.T
    wtail = jnp.pad(W[V - 64:].T, ((0, 0), (0, 64)))
    wp = _relayout_call(V, D)(wt, wtail)
    o5 = _gather_call(B, V, D, B0, S)(xf, wp)
    return o5.transpose(2, 4, 0, 1, 3).reshape(B0, S, D)
